# Initial kernel scaffold; baseline (speedup 1.0000x reference)
import jax
import jax.numpy as jnp
from jax.experimental import pallas as pl


def kernel(adj_edge_index, adj_values, sg1_edge_index, sg1_values, sg2_edge_index, sg2_values, users, items, neg_items, user_emb, item_emb):
    raise NotImplementedError("write your pallas kernel here")


# speedup vs baseline: 4.7752x; 4.7752x over previous
import functools
import jax
import jax.numpy as jnp
from jax import lax
from jax.experimental import pallas as pl
from jax.experimental.pallas import tpu as pltpu
from jax.experimental.pallas import tpu_sc as plsc

N_USERS = 25000
N_NODES = 50000
NCAP = 51000
D = 64
HD = 32
NL = 3
NS = 16
CHUNK = 512
GRP = 128
NGRP = CHUNK // GRP
TROWS = 3128
ACC_ROWS = NS * TROWS


NBSUB = 64


def _make_spmm_kernel(n_chunks_per_tile, nb_total):
    mesh = plsc.VectorSubcoreMesh(core_axis_name="c", subcore_axis_name="s")
    out_t = (
        jax.ShapeDtypeStruct((2 * NCAP, HD), jnp.float32),
        jax.ShapeDtypeStruct((2 * NCAP, HD), jnp.float32),
        jax.ShapeDtypeStruct((2 * NCAP, HD), jnp.float32),
        jax.ShapeDtypeStruct((2 * nb_total, HD), jnp.float32),
    )
    scratch = [
        pltpu.VMEM_SHARED((ACC_ROWS, HD), jnp.float32),
        pltpu.VMEM((NGRP, GRP), jnp.int32),
        pltpu.VMEM((NGRP, GRP), jnp.int32),
        pltpu.VMEM((NGRP, GRP), jnp.float32),
        pltpu.VMEM((CHUNK, HD), jnp.float32),
        pltpu.VMEM((NBSUB,), jnp.int32),
        pltpu.VMEM((4 * NBSUB, HD), jnp.float32),
        pltpu.SemaphoreType.DMA,
    ]

    @functools.partial(pl.kernel, out_type=out_t, scratch_types=scratch,
                       mesh=mesh,
                       compiler_params=pltpu.CompilerParams(
                           use_tc_tiling_on_sc=False))
    def k(rows2, cols2, vals2, x0, zrows, bidx, o1, o2, o3, ob,
          acc, rowbuf, colbuf, valbuf, gbuf, bidxbuf, b4, sem):
        c = lax.axis_index("c")
        s = lax.axis_index("s")
        coff = c * NCAP

        tables = [x0, o1, o2]
        outs = [o1, o2, o3]
        for layer in range(NL):
            src = tables[layer]
            dst = outs[layer]
            pltpu.sync_copy(zrows, acc.at[pl.ds(s * TROWS, TROWS)])
            plsc.subcore_barrier()

            def chunk_body(i, carry):
                base = (s * n_chunks_per_tile + i) * NGRP
                pltpu.sync_copy(rows2.at[pl.ds(base, NGRP)], rowbuf)
                pltpu.sync_copy(cols2.at[pl.ds(base, NGRP)], colbuf)
                pltpu.sync_copy(vals2.at[pl.ds(base, NGRP)], valbuf)
                for j in range(NGRP):
                    for q in range(GRP // 16):
                        sl = pl.ds(q * 16, 16)
                        colbuf[j, sl] = colbuf[j, sl] + coff
                cps = [pltpu.async_copy(src.at[colbuf.at[j]],
                                        gbuf.at[pl.ds(j * GRP, GRP)], sem)
                       for j in range(NGRP)]
                for cp in cps:
                    cp.wait()

                def sgrp(g, carry2):
                    jj = g // (GRP // 16)
                    qq = lax.rem(g, GRP // 16)
                    vv16 = valbuf[jj, pl.ds(qq * 16, 16)]
                    for lane in range(16):
                        e = g * 16 + lane
                        vb = jnp.full((16,), vv16[lane], jnp.float32)
                        gbuf[e, pl.ds(0, 16)] = gbuf[e, pl.ds(0, 16)] * vb
                        gbuf[e, pl.ds(16, 16)] = gbuf[e, pl.ds(16, 16)] * vb
                    return carry2
                lax.fori_loop(0, CHUNK // 16, sgrp, 0)

                for j in range(NGRP):
                    pltpu.sync_copy(gbuf.at[pl.ds(j * GRP, GRP)],
                                    acc.at[rowbuf.at[j]], add=True)
                return carry
            lax.fori_loop(0, n_chunks_per_tile, chunk_body, 0)
            plsc.subcore_barrier()
            pltpu.sync_copy(acc.at[pl.ds(s * TROWS, TROWS)],
                            dst.at[pl.ds(coff + s * TROWS, TROWS)])
            plsc.subcore_barrier()

        npass = nb_total // (NS * NBSUB)
        for p in range(npass):
            b0 = (s * npass + p) * NBSUB
            pltpu.sync_copy(bidx.at[pl.ds(b0, NBSUB)], bidxbuf)
            for q in range(NBSUB // 16):
                sl = pl.ds(q * 16, 16)
                bidxbuf[sl] = bidxbuf[sl] + coff
            cps = [pltpu.async_copy(
                       tab.at[bidxbuf],
                       b4.at[pl.ds(t * NBSUB, NBSUB)], sem)
                   for t, tab in enumerate([x0, o1, o2, o3])]
            for cp in cps:
                cp.wait()

            def addrow(e, carry2):
                for sl in (pl.ds(0, 16), pl.ds(16, 16)):
                    b4[e, sl] = ((b4[e, sl] + b4[NBSUB + e, sl])
                                 + (b4[2 * NBSUB + e, sl]
                                    + b4[3 * NBSUB + e, sl]))
                return carry2
            lax.fori_loop(0, NBSUB, addrow, 0, unroll=4)
            pltpu.sync_copy(b4.at[pl.ds(0, NBSUB)],
                            ob.at[pl.ds(c * nb_total + b0, NBSUB)])

    return k


def _prep_body(adjb, sg1b, sg2b, sup, u1b, i1b, posu, posi):
    def half(ref, lo):
        return jnp.concatenate([ref[lo:lo + 1024, :],
                                ref[ref.shape[0] // 2 + lo:
                                    ref.shape[0] // 2 + lo + 1024, :]],
                               axis=1)
    ue = half(adjb, 0) * 0.25
    ie = half(adjb, 1024) * 0.25
    ien = half(adjb, 2048) * 0.25
    pos = jnp.sum(ue * ie, axis=1, keepdims=True)
    neg = jnp.sum(ue * ien, axis=1, keepdims=True)
    sup[...] = pos - neg

    def norm_rows(r):
        n = jnp.sqrt(jnp.sum(r * r, axis=1, keepdims=True))
        return r / jnp.maximum(n, 4e-12)
    u1 = norm_rows(half(sg1b, 0))
    i1 = norm_rows(half(sg1b, 1024))
    u2 = norm_rows(half(sg2b, 0))
    i2 = norm_rows(half(sg2b, 1024))
    u1b[...] = u1
    i1b[...] = i1
    posu[...] = jnp.sum(u1 * u2, axis=1, keepdims=True)
    posi[...] = jnp.sum(i1 * i2, axis=1, keepdims=True)


def _prep(adjb, sg1b, sg2b):
    return pl.pallas_call(
        _prep_body,
        out_shape=(
            jax.ShapeDtypeStruct((1024, 1), jnp.float32),
            jax.ShapeDtypeStruct((1024, D), jnp.float32),
            jax.ShapeDtypeStruct((1024, D), jnp.float32),
            jax.ShapeDtypeStruct((1024, 1), jnp.float32),
            jax.ShapeDtypeStruct((1024, 1), jnp.float32),
        ),
    )(adjb, sg1b, sg2b)


_BN = 1000


def _norm_body(x0l, x1l, x2l, x3l, x0h, x1h, x2h, x3h, out):
    lo = x0l[...] + x1l[...] + x2l[...] + x3l[...]
    hi = x0h[...] + x1h[...] + x2h[...] + x3h[...]
    ss = (jnp.sum(lo * lo, axis=1, keepdims=True)
          + jnp.sum(hi * hi, axis=1, keepdims=True))
    inv = 1.0 / jnp.maximum(jnp.sqrt(ss), 4e-12)
    out[...] = jnp.concatenate([lo * inv, hi * inv], axis=1)


def _norm_full(x0, t1, t2, t3):
    lospec = pl.BlockSpec((_BN, HD), lambda j: (j, 0))
    hispec = pl.BlockSpec((_BN, HD), lambda j: (NCAP // _BN + j, 0))
    return pl.pallas_call(
        _norm_body,
        grid=(N_NODES // _BN,),
        in_specs=[lospec, lospec, lospec, lospec,
                  hispec, hispec, hispec, hispec],
        out_specs=pl.BlockSpec((_BN, D), lambda j: (j, 0)),
        out_shape=jax.ShapeDtypeStruct((N_NODES, D), jnp.float32),
        compiler_params=pltpu.CompilerParams(
            dimension_semantics=("arbitrary",)),
    )(x0, t1, t2, t3, x0, t1, t2, t3)


def _ssl_body(ub, pos, tab, out):
    out[...] = jnp.dot(ub[...], tab[...].T,
                       preferred_element_type=jnp.float32) - pos[...]


def _ssl(ub, pos, table, side):
    return pl.pallas_call(
        _ssl_body,
        grid=(1024 // 128,),
        in_specs=[
            pl.BlockSpec((128, D), lambda i: (i, 0)),
            pl.BlockSpec((128, 1), lambda i: (i, 0)),
            pl.BlockSpec((N_USERS, D), lambda i: (side, 0)),
        ],
        out_specs=pl.BlockSpec((128, N_USERS), lambda i: (i, 0)),
        out_shape=jax.ShapeDtypeStruct((1024, N_USERS), jnp.float32),
        compiler_params=pltpu.CompilerParams(
            dimension_semantics=("arbitrary",)),
    )(ub, pos, table)


def _prep_edges(ei, v, n_chunks_per_tile):
    ep = NS * CHUNK * n_chunks_per_tile
    e = v.shape[0]
    npad = ep - e
    pad_ids = (jnp.arange(npad, dtype=jnp.int32) * 64) % N_NODES
    r = jnp.concatenate([ei[0], pad_ids]).reshape(-1, GRP)
    cc = jnp.concatenate([ei[1], pad_ids]).reshape(-1, GRP)
    vv = jnp.concatenate([v, jnp.zeros((npad,), jnp.float32)]).reshape(-1, GRP)
    return r, cc, vv


def kernel(adj_edge_index, adj_values, sg1_edge_index, sg1_values,
           sg2_edge_index, sg2_values, users, items, neg_items,
           user_emb, item_emb):
    ego = jnp.concatenate([user_emb, item_emb], axis=0)
    zpad = jnp.zeros((NCAP - N_NODES, HD), jnp.float32)
    x0 = jnp.concatenate([ego[:, :HD], zpad, ego[:, HD:], zpad], axis=0)
    zrows = jnp.zeros((TROWS, HD), jnp.float32)
    adj_bidx = jnp.concatenate([users, items + N_USERS, neg_items + N_USERS])
    sg_bidx = jnp.concatenate([users, items + N_USERS])

    n_adj = -(-adj_values.shape[0] // (NS * CHUNK))
    n_sg = -(-sg1_values.shape[0] // (NS * CHUNK))
    ar, ac, av = _prep_edges(adj_edge_index, adj_values, n_adj)
    s1r, s1c, s1v = _prep_edges(sg1_edge_index, sg1_values, n_sg)
    s2r, s2c, s2v = _prep_edges(sg2_edge_index, sg2_values, n_sg)

    k_adj = _make_spmm_kernel(n_adj, 3072)
    k_sg = _make_spmm_kernel(n_sg, 2048)
    _, _, _, adj_b = k_adj(ar, ac, av, x0, zrows, adj_bidx)
    _, _, _, sg1_b = k_sg(s1r, s1c, s1v, x0, zrows, sg_bidx)
    t1, t2, t3, sg2_b = k_sg(s2r, s2c, s2v, x0, zrows, sg_bidx)

    sup, u1b, i1b, posu, posi = _prep(adj_b, sg1_b, sg2_b)
    m2 = _norm_full(x0, t1, t2, t3)
    ssl_u = _ssl(u1b, posu, m2, 0)
    ssl_i = _ssl(i1b, posi, m2, 1)
    return sup.reshape(1024), ssl_u, ssl_i


# speedup vs baseline: 7.2267x; 1.5134x over previous
import functools
import jax
import jax.numpy as jnp
from jax import lax
from jax.experimental import pallas as pl
from jax.experimental.pallas import tpu as pltpu
from jax.experimental.pallas import tpu_sc as plsc

N_USERS = 25000
N_NODES = 50000
NCAP = 51000
D = 64
HD = 32
NL = 3
NS = 16
CHUNK = 192
ROT = 4
TROWS = 3128
ACC_ROWS = NS * TROWS
NBSUB = 16


def _make_spmm_kernel(n_chunks_per_tile, nb_total):
    n = n_chunks_per_tile
    assert n % ROT == 0
    mesh = plsc.VectorSubcoreMesh(core_axis_name="c", subcore_axis_name="s")
    out_t = (
        jax.ShapeDtypeStruct((2 * NCAP, HD), jnp.float32),
        jax.ShapeDtypeStruct((2 * NCAP, HD), jnp.float32),
        jax.ShapeDtypeStruct((2 * NCAP, HD), jnp.float32),
        jax.ShapeDtypeStruct((2 * nb_total, HD), jnp.float32),
    )
    scratch = (
        [pltpu.VMEM_SHARED((ACC_ROWS, HD), jnp.float32)]
        + [pltpu.VMEM((3, CHUNK), jnp.int32) for _ in range(ROT)]
        + [pltpu.VMEM((CHUNK, HD), jnp.float32) for _ in range(ROT)]
        + [pltpu.VMEM((CHUNK,), jnp.int32) for _ in range(ROT)]
        + [pltpu.VMEM((NBSUB,), jnp.int32),
           pltpu.VMEM((4 * NBSUB, HD), jnp.float32)]
        + [pltpu.SemaphoreType.DMA for _ in range(3 * ROT)]
    )

    @functools.partial(pl.kernel, out_type=out_t, scratch_types=scratch,
                       mesh=mesh,
                       compiler_params=pltpu.CompilerParams(
                           use_tc_tiling_on_sc=False,
                           needs_layout_passes=False))
    def k(einter, x0, zrows, bidx, o1, o2, o3, ob, *scr):
        acc = scr[0]
        ebuf = scr[1:1 + ROT]
        gbuf = scr[1 + ROT:1 + 2 * ROT]
        srow = scr[1 + 2 * ROT:1 + 3 * ROT]
        bidxbuf = scr[1 + 3 * ROT]
        b4 = scr[2 + 3 * ROT]
        esem = scr[3 + 3 * ROT:3 + 4 * ROT]
        gsem = scr[3 + 4 * ROT:3 + 5 * ROT]
        ssem = scr[3 + 5 * ROT:3 + 6 * ROT]

        c = lax.axis_index("c")
        s = lax.axis_index("s")
        coff = c * NCAP
        base = s * n

        def offset_cols(b):
            for q in range(CHUNK // 16):
                sl = pl.ds(q * 16, 16)
                ebuf[b][1, sl] = ebuf[b][1, sl] + coff

        def srow_copy(b):
            for q in range(CHUNK // 16):
                sl = pl.ds(q * 16, 16)
                srow[b][sl] = ebuf[b][0, sl]

        def stage_start(gc, b):
            pltpu.async_copy(einter.at[gc], ebuf[b], esem[b])

        def stage_wait(b):
            pltpu.make_async_copy(einter.at[0], ebuf[b], esem[b]).wait()

        def gather_start(src_tab, b):
            pltpu.async_copy(src_tab.at[ebuf[b].at[1]], gbuf[b], gsem[b])

        def gather_wait(src_tab, b):
            pltpu.make_async_copy(src_tab.at[ebuf[b].at[1]], gbuf[b],
                                  gsem[b]).wait()

        def scatter_start(b):
            pltpu.async_copy(gbuf[b], acc.at[srow[b]], ssem[b], add=True)

        def scatter_wait(b):
            pltpu.make_async_copy(gbuf[b], acc.at[srow[b]], ssem[b]).wait()

        def scale(b):
            def sgrp(q, carry2):
                vv16 = plsc.bitcast(ebuf[b][2, pl.ds(q * 16, 16)],
                                    jnp.float32)
                for lane in range(16):
                    e = q * 16 + lane
                    vb = jnp.full((16,), vv16[lane], jnp.float32)
                    gbuf[b][e, pl.ds(0, 16)] = gbuf[b][e, pl.ds(0, 16)] * vb
                    gbuf[b][e, pl.ds(16, 16)] = (gbuf[b][e, pl.ds(16, 16)]
                                                 * vb)
                return carry2
            lax.fori_loop(0, CHUNK // 16, sgrp, 0)

        tables = [x0, o1, o2]
        outs = [o1, o2, o3]
        for layer in range(NL):
            src_tab = tables[layer]
            dst = outs[layer]
            pltpu.sync_copy(zrows, acc.at[pl.ds(s * TROWS, TROWS)])
            plsc.subcore_barrier()

            for cc in range(3):
                stage_start(base + cc, cc)
            for cc in range(2):
                stage_wait(cc)
                offset_cols(cc)
                gather_start(src_tab, cc)

            def body(i4, carry):
                for kk in range(ROT):
                    ch = i4 * ROT + kk
                    bX = kk
                    bZ = (kk + 2) % ROT
                    bS = (kk + 3) % ROT

                    @pl.when(ch + 2 < n)
                    def _():
                        stage_wait(bZ)
                        offset_cols(bZ)

                    @pl.when(ch >= 2)
                    def _():
                        scatter_wait(bZ)

                    @pl.when(ch + 2 < n)
                    def _():
                        gather_start(src_tab, bZ)

                    gather_wait(src_tab, bX)
                    scale(bX)
                    srow_copy(bX)
                    scatter_start(bX)

                    @pl.when(ch + 3 < n)
                    def _():
                        stage_start(base + ch + 3, bS)
                return carry
            lax.fori_loop(0, n // ROT, body, 0)
            for cc in ((n - 2) % ROT, (n - 1) % ROT):
                scatter_wait(cc)
            plsc.subcore_barrier()
            pltpu.sync_copy(acc.at[pl.ds(s * TROWS, TROWS)],
                            dst.at[pl.ds(coff + s * TROWS, TROWS)])
            plsc.subcore_barrier()

        sem = esem[0]
        npass = nb_total // (NS * NBSUB)
        for p in range(npass):
            b0 = (s * npass + p) * NBSUB
            pltpu.sync_copy(bidx.at[pl.ds(b0, NBSUB)], bidxbuf)
            for q in range(NBSUB // 16):
                sl = pl.ds(q * 16, 16)
                bidxbuf[sl] = bidxbuf[sl] + coff
            cps = [pltpu.async_copy(
                       tab.at[bidxbuf],
                       b4.at[pl.ds(t * NBSUB, NBSUB)], sem)
                   for t, tab in enumerate([x0, o1, o2, o3])]
            for cp in cps:
                cp.wait()

            def addrow(e, carry2):
                for sl in (pl.ds(0, 16), pl.ds(16, 16)):
                    b4[e, sl] = ((b4[e, sl] + b4[NBSUB + e, sl])
                                 + (b4[2 * NBSUB + e, sl]
                                    + b4[3 * NBSUB + e, sl]))
                return carry2
            lax.fori_loop(0, NBSUB, addrow, 0, unroll=4)
            pltpu.sync_copy(b4.at[pl.ds(0, NBSUB)],
                            ob.at[pl.ds(c * nb_total + b0, NBSUB)])

    return k


def _prep_body(adjb, sg1b, sg2b, sup, u1b, i1b, posu, posi):
    def half(ref, lo):
        return jnp.concatenate([ref[lo:lo + 1024, :],
                                ref[ref.shape[0] // 2 + lo:
                                    ref.shape[0] // 2 + lo + 1024, :]],
                               axis=1)
    ue = half(adjb, 0) * 0.25
    ie = half(adjb, 1024) * 0.25
    ien = half(adjb, 2048) * 0.25
    pos = jnp.sum(ue * ie, axis=1, keepdims=True)
    neg = jnp.sum(ue * ien, axis=1, keepdims=True)
    sup[...] = pos - neg

    def norm_rows(r):
        n = jnp.sqrt(jnp.sum(r * r, axis=1, keepdims=True))
        return r / jnp.maximum(n, 4e-12)
    u1 = norm_rows(half(sg1b, 0))
    i1 = norm_rows(half(sg1b, 1024))
    u2 = norm_rows(half(sg2b, 0))
    i2 = norm_rows(half(sg2b, 1024))
    u1b[...] = u1
    i1b[...] = i1
    posu[...] = jnp.sum(u1 * u2, axis=1, keepdims=True)
    posi[...] = jnp.sum(i1 * i2, axis=1, keepdims=True)


def _prep(adjb, sg1b, sg2b):
    return pl.pallas_call(
        _prep_body,
        out_shape=(
            jax.ShapeDtypeStruct((1024, 1), jnp.float32),
            jax.ShapeDtypeStruct((1024, D), jnp.float32),
            jax.ShapeDtypeStruct((1024, D), jnp.float32),
            jax.ShapeDtypeStruct((1024, 1), jnp.float32),
            jax.ShapeDtypeStruct((1024, 1), jnp.float32),
        ),
    )(adjb, sg1b, sg2b)


_BN = 1000


def _norm_body(x0l, x1l, x2l, x3l, x0h, x1h, x2h, x3h, out):
    lo = x0l[...] + x1l[...] + x2l[...] + x3l[...]
    hi = x0h[...] + x1h[...] + x2h[...] + x3h[...]
    ss = (jnp.sum(lo * lo, axis=1, keepdims=True)
          + jnp.sum(hi * hi, axis=1, keepdims=True))
    inv = 1.0 / jnp.maximum(jnp.sqrt(ss), 4e-12)
    out[...] = jnp.concatenate([lo * inv, hi * inv], axis=1)


def _norm_full(x0, t1, t2, t3):
    lospec = pl.BlockSpec((_BN, HD), lambda j: (j, 0))
    hispec = pl.BlockSpec((_BN, HD), lambda j: (NCAP // _BN + j, 0))
    return pl.pallas_call(
        _norm_body,
        grid=(N_NODES // _BN,),
        in_specs=[lospec, lospec, lospec, lospec,
                  hispec, hispec, hispec, hispec],
        out_specs=pl.BlockSpec((_BN, D), lambda j: (j, 0)),
        out_shape=jax.ShapeDtypeStruct((N_NODES, D), jnp.float32),
        compiler_params=pltpu.CompilerParams(
            dimension_semantics=("arbitrary",)),
    )(x0, t1, t2, t3, x0, t1, t2, t3)


def _ssl_body(ub, pos, tab, out):
    out[...] = jnp.dot(ub[...], tab[...].T,
                       preferred_element_type=jnp.float32) - pos[...]


def _ssl(ub, pos, table, side):
    return pl.pallas_call(
        _ssl_body,
        grid=(1024 // 128,),
        in_specs=[
            pl.BlockSpec((128, D), lambda i: (i, 0)),
            pl.BlockSpec((128, 1), lambda i: (i, 0)),
            pl.BlockSpec((N_USERS, D), lambda i: (side, 0)),
        ],
        out_specs=pl.BlockSpec((128, N_USERS), lambda i: (i, 0)),
        out_shape=jax.ShapeDtypeStruct((1024, N_USERS), jnp.float32),
        compiler_params=pltpu.CompilerParams(
            dimension_semantics=("arbitrary",)),
    )(ub, pos, table)


def _prep_edges(ei, v, n_chunks_per_tile):
    ep = NS * CHUNK * n_chunks_per_tile
    npad = ep - v.shape[0]
    pad_ids = (jnp.arange(npad, dtype=jnp.int32) * 64) % N_NODES
    r = jnp.concatenate([ei[0], pad_ids]).reshape(-1, 1, CHUNK)
    cc = jnp.concatenate([ei[1], pad_ids]).reshape(-1, 1, CHUNK)
    vi = lax.bitcast_convert_type(
        jnp.concatenate([v, jnp.zeros((npad,), jnp.float32)]),
        jnp.int32).reshape(-1, 1, CHUNK)
    return jnp.concatenate([r, cc, vi], axis=1)


def kernel(adj_edge_index, adj_values, sg1_edge_index, sg1_values,
           sg2_edge_index, sg2_values, users, items, neg_items,
           user_emb, item_emb):
    ego = jnp.concatenate([user_emb, item_emb], axis=0)
    zpad = jnp.zeros((NCAP - N_NODES, HD), jnp.float32)
    x0 = jnp.concatenate([ego[:, :HD], zpad, ego[:, HD:], zpad], axis=0)
    zrows = jnp.zeros((TROWS, HD), jnp.float32)
    adj_bidx = jnp.concatenate([users, items + N_USERS, neg_items + N_USERS])
    sg_bidx = jnp.concatenate([users, items + N_USERS])

    def nchunks(e):
        nc = -(-e // (NS * CHUNK))
        return -(-nc // ROT) * ROT
    n_adj = nchunks(adj_values.shape[0])
    n_sg = nchunks(sg1_values.shape[0])
    ea = _prep_edges(adj_edge_index, adj_values, n_adj)
    e1 = _prep_edges(sg1_edge_index, sg1_values, n_sg)
    e2 = _prep_edges(sg2_edge_index, sg2_values, n_sg)

    k_adj = _make_spmm_kernel(n_adj, 3072)
    k_sg = _make_spmm_kernel(n_sg, 2048)
    _, _, _, adj_b = k_adj(ea, x0, zrows, adj_bidx)
    _, _, _, sg1_b = k_sg(e1, x0, zrows, sg_bidx)
    t1, t2, t3, sg2_b = k_sg(e2, x0, zrows, sg_bidx)

    sup, u1b, i1b, posu, posi = _prep(adj_b, sg1_b, sg2_b)
    m2 = _norm_full(x0, t1, t2, t3)
    ssl_u = _ssl(u1b, posu, m2, 0)
    ssl_i = _ssl(i1b, posi, m2, 1)
    return sup.reshape(1024), ssl_u, ssl_i
